# baseline (speedup 1.0000x reference)
import functools
import jax
import jax.numpy as jnp
from jax import lax
from jax.experimental import pallas as pl
from jax.experimental.pallas import tpu as pltpu, tpu_sc as plsc

B, N, NP, K, CIN, COUT, P = 8, 8192, 2048, 32, 64, 64, 5
RADIUS = 1.5
EPS = 1e-5

NC, NS = 2, 16
NW = NC * NS
NCENT = B * NP
CPW = NCENT // NW
CHUNK_C = 128
NCHUNK = CPW // CHUNK_C
ROWS_PER_CHUNK = CHUNK_C * K
BLK = 128
NBLK = ROWS_PER_CHUNK // BLK
GRP = 2
NGRP = NBLK // GRP
ACC_ROWS = CHUNK_C * P
NREG = 2
ZROWS = 64
NXPW = NCENT // NW
NXBLK = NXPW // BLK


def _idx_kernel(xcol, ycol, neigh, didx, src_out, dst_out, nx_out):
    j = pl.program_id(0)
    rows = xcol.shape[0]
    base = j * rows * 128
    ii = (base
          + lax.broadcasted_iota(jnp.int32, (rows, 128), 0) * 128
          + lax.broadcasted_iota(jnp.int32, (rows, 128), 1))
    dist = jnp.sqrt(xcol[...] * xcol[...] + ycol[...] * ycol[...])
    dist = jnp.minimum(dist / RADIUS, 0.99)
    bins = jnp.floor(dist * P).astype(jnp.int32)
    b = ii >> 16
    src_out[...] = b * N + neigh[...]
    c = ii >> 5
    s = c >> 10
    reg = (c >> 7) & 1
    cl = c & (CHUNK_C - 1)
    dst_out[...] = (s * NREG + reg) * ACC_ROWS + cl * P + bins
    nrows = didx.shape[0]
    i2 = (j * nrows * 128
          + lax.broadcasted_iota(jnp.int32, (nrows, 128), 0) * 128
          + lax.broadcasted_iota(jnp.int32, (nrows, 128), 1))
    b2 = i2 >> 11
    nx_out[...] = b2 * N + didx[...]


def _make_indices(lc, neighbor_lists, data_idx):
    lcf = lc.reshape(NCENT * K, 3)
    xcol = lcf[:, 0].reshape(NCENT * K // 128, 128)
    ycol = lcf[:, 1].reshape(NCENT * K // 128, 128)
    neigh = neighbor_lists.reshape(NCENT * K // 128, 128).astype(jnp.int32)
    didx = data_idx.reshape(NCENT // 128, 128).astype(jnp.int32)
    G = 4
    rb = NCENT * K // 128 // G
    nb = NCENT // 128 // G
    src, dst, nx = pl.pallas_call(
        _idx_kernel,
        grid=(G,),
        in_specs=[
            pl.BlockSpec((rb, 128), lambda j: (j, 0)),
            pl.BlockSpec((rb, 128), lambda j: (j, 0)),
            pl.BlockSpec((rb, 128), lambda j: (j, 0)),
            pl.BlockSpec((nb, 128), lambda j: (j, 0)),
        ],
        out_specs=[
            pl.BlockSpec((rb, 128), lambda j: (j, 0)),
            pl.BlockSpec((rb, 128), lambda j: (j, 0)),
            pl.BlockSpec((nb, 128), lambda j: (j, 0)),
        ],
        out_shape=[
            jax.ShapeDtypeStruct((NCENT * K // 128, 128), jnp.int32),
            jax.ShapeDtypeStruct((NCENT * K // 128, 128), jnp.int32),
            jax.ShapeDtypeStruct((NCENT // 128, 128), jnp.int32),
        ],
    )(xcol, ycol, neigh, didx)
    return src, dst, nx


def _sc_body(pts_hbm, src_hbm, dst_hbm, nx_hbm, zeros_hbm, feat_hbm, nxr_hbm,
             src_v, dst_v, bufA, bufB, nxi_v, acc_sh,
             semA, semB, semW0, semW1):
    s = lax.axis_index("s")
    c = lax.axis_index("c")
    w = s * NC + c
    dummy_grp = pts_hbm.at[pl.ds(0, GRP * BLK)]

    def zero_region(reg):
        pltpu.sync_copy(
            zeros_hbm, acc_sh.at[pl.ds((s * NREG + reg) * ACC_ROWS, ACC_ROWS)])

    zero_region(0)
    zero_region(1)
    semW = (semW0, semW1)

    def fire_group(g, buf, sem, cbase):
        for k in range(GRP):
            pltpu.async_copy(pts_hbm.at[src_v.at[cbase + g * GRP + k]],
                             buf.at[pl.ds(k * BLK, BLK)], sem)

    for chunk in range(NCHUNK):
        reg = chunk % NREG
        accbase = (s * NREG + reg) * ACC_ROWS
        cbase = 0
        pltpu.sync_copy(src_hbm.at[pl.ds((w * NCHUNK + chunk) * NBLK, NBLK)],
                        src_v)
        pltpu.sync_copy(dst_hbm.at[pl.ds((w * NCHUNK + chunk) * NBLK, NBLK)],
                        dst_v)
        wb_dst = feat_hbm.at[pl.ds((w * NCHUNK + chunk) * ACC_ROWS, ACC_ROWS)]
        if chunk >= NREG:
            pltpu.make_async_copy(
                acc_sh.at[pl.ds(accbase, ACC_ROWS)], wb_dst, semW[reg]).wait()
            zero_region(reg)

        fire_group(0, bufA, semA, cbase)

        def pair_body(i, _):
            gA = 2 * i
            pltpu.make_async_copy(dummy_grp, bufA, semA).wait()
            fire_group(gA + 1, bufB, semB, cbase)
            for k in range(GRP):
                pltpu.sync_copy(bufA.at[pl.ds(k * BLK, BLK)],
                                acc_sh.at[dst_v.at[cbase + gA * GRP + k]],
                                add=True)
            pltpu.make_async_copy(dummy_grp, bufB, semB).wait()
            fire_group(lax.rem(gA + 2, NGRP), bufA, semA, cbase)
            for k in range(GRP):
                pltpu.sync_copy(bufB.at[pl.ds(k * BLK, BLK)],
                                acc_sh.at[dst_v.at[cbase + (gA + 1) * GRP + k]],
                                add=True)
            return 0
        lax.fori_loop(0, NGRP // 2, pair_body, 0)
        pltpu.make_async_copy(dummy_grp, bufA, semA).wait()

        pltpu.async_copy(acc_sh.at[pl.ds(accbase, ACC_ROWS)], wb_dst, semW[reg])

    for chunk in range(NCHUNK - NREG, NCHUNK):
        reg = chunk % NREG
        accbase = (s * NREG + reg) * ACC_ROWS
        wb_dst = feat_hbm.at[pl.ds((w * NCHUNK + chunk) * ACC_ROWS, ACC_ROWS)]
        pltpu.make_async_copy(
            acc_sh.at[pl.ds(accbase, ACC_ROWS)], wb_dst, semW[reg]).wait()

    pltpu.sync_copy(nx_hbm.at[pl.ds(w * NXBLK, NXBLK)], nxi_v)

    def nx_body(m, _):
        pltpu.async_copy(pts_hbm.at[nxi_v.at[m]],
                         bufA.at[pl.ds(0, BLK)], semA).wait()
        pltpu.sync_copy(bufA.at[pl.ds(0, BLK)],
                        nxr_hbm.at[pl.ds(w * NXPW + m * BLK, BLK)])
        return 0
    lax.fori_loop(0, NXBLK, nx_body, 0)


def _sc_gather_scatter(pts_flat, src_idx, dst_idx, nx_idx, zeros):
    mesh = plsc.VectorSubcoreMesh(core_axis_name="c", subcore_axis_name="s")
    fn = pl.kernel(
        _sc_body,
        out_type=[
            jax.ShapeDtypeStruct((NCENT * P, CIN), jnp.float32),
            jax.ShapeDtypeStruct((NCENT, CIN), jnp.float32),
        ],
        mesh=mesh,
        scratch_types=[
            pltpu.VMEM((NBLK, BLK), jnp.int32),
            pltpu.VMEM((NBLK, BLK), jnp.int32),
            pltpu.VMEM((GRP * BLK, CIN), jnp.float32),
            pltpu.VMEM((GRP * BLK, CIN), jnp.float32),
            pltpu.VMEM((NXBLK, BLK), jnp.int32),
            pltpu.VMEM_SHARED((NS * NREG * ACC_ROWS, CIN), jnp.float32),
            pltpu.SemaphoreType.DMA,
            pltpu.SemaphoreType.DMA,
            pltpu.SemaphoreType.DMA,
            pltpu.SemaphoreType.DMA,
        ],
        compiler_params=pltpu.CompilerParams(use_tc_tiling_on_sc=False),
    )
    return fn(pts_flat, src_idx, dst_idx, nx_idx, zeros)


def _head_kernel(feat, wc, bc, g1, be1, wl, bl, g2, be2, out):
    n = jnp.float32(NCENT)
    x = lax.dot_general(feat[...], wc[...], (((1,), (1,)), ((), ())),
                        preferred_element_type=jnp.float32) + bc[...]
    mu = jnp.sum(x, axis=0, keepdims=True) / n
    var = jnp.sum(x * x, axis=0, keepdims=True) / n - mu * mu
    x = (x - mu) / jnp.sqrt(var + EPS) * g1[...] + be1[...]
    x = jnp.maximum(x, 0.0)
    x = lax.dot_general(x, wl[...], (((1,), (1,)), ((), ())),
                        preferred_element_type=jnp.float32) + bl[...]
    mu2 = jnp.sum(x, axis=0, keepdims=True) / n
    var2 = jnp.sum(x * x, axis=0, keepdims=True) / n - mu2 * mu2
    x = (x - mu2) / jnp.sqrt(var2 + EPS) * g2[...] + be2[...]
    out[...] = jnp.maximum(x, 0.0)


def _head(feat, W_conv, b_conv, gamma1, beta1, W_lin, b_lin, gamma2, beta2):
    return pl.pallas_call(
        _head_kernel,
        out_shape=jax.ShapeDtypeStruct((NCENT, COUT), jnp.float32),
    )(feat, W_conv, b_conv.reshape(1, COUT), gamma1.reshape(1, COUT),
      beta1.reshape(1, COUT), W_lin, b_lin.reshape(1, COUT),
      gamma2.reshape(1, COUT), beta2.reshape(1, COUT))


@jax.jit
def _run(xyz, points, local_coordinates, neighbor_lists, data_idx,
         W_conv, b_conv, gamma1, beta1, W_lin, b_lin, gamma2, beta2):
    pts_flat = jnp.concatenate([points, xyz], axis=2).reshape(B * N, CIN)
    src_idx, dst_idx, nx_idx = _make_indices(
        local_coordinates, neighbor_lists, data_idx)
    zeros = jnp.zeros((ACC_ROWS, CIN), jnp.float32)
    feat_rows, nx_rows = _sc_gather_scatter(pts_flat, src_idx, dst_idx,
                                            nx_idx, zeros)
    feat = feat_rows.reshape(NCENT, P * CIN)
    out = _head(feat, W_conv, b_conv, gamma1, beta1, W_lin, b_lin,
                gamma2, beta2)
    new_xyz = nx_rows[:, CIN - 3:].reshape(B, NP, 3)
    new_points = out.reshape(B, NP, COUT)
    return new_xyz, new_points


def kernel(xyz, points, local_coordinates, neighbor_lists, parameter_list,
           data_idx, W_conv, b_conv, gamma1, beta1, W_lin, b_lin,
           gamma2, beta2):
    return _run(xyz, points, local_coordinates, neighbor_lists, data_idx,
                W_conv, b_conv, gamma1, beta1, W_lin, b_lin, gamma2, beta2)


# speedup vs baseline: 1.0812x; 1.0812x over previous
import functools
import jax
import jax.numpy as jnp
from jax import lax
from jax.experimental import pallas as pl
from jax.experimental.pallas import tpu as pltpu, tpu_sc as plsc

B, N, NP, K, CIN, COUT, P = 8, 8192, 2048, 32, 64, 64, 5
RADIUS = 1.5
EPS = 1e-5

NC, NS = 2, 16
NW = NC * NS
NCENT = B * NP
CPW = NCENT // NW
CHUNK_C = 64
NCHUNK = CPW // CHUNK_C
ROWS_PER_CHUNK = CHUNK_C * K
BLK = 128
NBLK = ROWS_PER_CHUNK // BLK
GRP = 2
NGRP = NBLK // GRP
NQ = 3
ACC_ROWS = CHUNK_C * 2 * NQ
PLANE_ROWS = CHUNK_C * 2
NREG = 2
ZROWS = 64
NXPW = NCENT // NW
NXBLK = NXPW // BLK


def _idx_kernel(xcol, ycol, neigh, didx, src_out, dst_out, nx_out):
    j = pl.program_id(0)
    rows = xcol.shape[0]
    base = j * rows * 128
    ii = (base
          + lax.broadcasted_iota(jnp.int32, (rows, 128), 0) * 128
          + lax.broadcasted_iota(jnp.int32, (rows, 128), 1))
    dist = jnp.sqrt(xcol[...] * xcol[...] + ycol[...] * ycol[...])
    dist = jnp.minimum(dist / RADIUS, 0.99)
    bins = jnp.floor(dist * P).astype(jnp.int32)
    b = ii >> 16
    src_out[...] = b * N + neigh[...]
    c = ii >> 5
    s = c >> 10
    reg = (c >> 6) & 1
    cl = c & (CHUNK_C - 1)
    dst_out[...] = ((s * NREG + reg) * ACC_ROWS + (bins >> 1) * PLANE_ROWS
                    + cl * 2 + (bins & 1))
    nrows = didx.shape[0]
    i2 = (j * nrows * 128
          + lax.broadcasted_iota(jnp.int32, (nrows, 128), 0) * 128
          + lax.broadcasted_iota(jnp.int32, (nrows, 128), 1))
    b2 = i2 >> 11
    nx_out[...] = b2 * N + didx[...]


def _make_indices(lc, neighbor_lists, data_idx):
    lcf = lc.reshape(NCENT * K, 3)
    xcol = lcf[:, 0].reshape(NCENT * K // 128, 128)
    ycol = lcf[:, 1].reshape(NCENT * K // 128, 128)
    neigh = neighbor_lists.reshape(NCENT * K // 128, 128).astype(jnp.int32)
    didx = data_idx.reshape(NCENT // 128, 128).astype(jnp.int32)
    G = 4
    rb = NCENT * K // 128 // G
    nb = NCENT // 128 // G
    src, dst, nx = pl.pallas_call(
        _idx_kernel,
        grid=(G,),
        in_specs=[
            pl.BlockSpec((rb, 128), lambda j: (j, 0)),
            pl.BlockSpec((rb, 128), lambda j: (j, 0)),
            pl.BlockSpec((rb, 128), lambda j: (j, 0)),
            pl.BlockSpec((nb, 128), lambda j: (j, 0)),
        ],
        out_specs=[
            pl.BlockSpec((rb, 128), lambda j: (j, 0)),
            pl.BlockSpec((rb, 128), lambda j: (j, 0)),
            pl.BlockSpec((nb, 128), lambda j: (j, 0)),
        ],
        out_shape=[
            jax.ShapeDtypeStruct((NCENT * K // 128, 128), jnp.int32),
            jax.ShapeDtypeStruct((NCENT * K // 128, 128), jnp.int32),
            jax.ShapeDtypeStruct((NCENT // 128, 128), jnp.int32),
        ],
    )(xcol, ycol, neigh, didx)
    return src, dst, nx


def _sc_body(pts_hbm, src_hbm, dst_hbm, nx_hbm, f0_hbm, f1_hbm, f2_hbm,
             nxr_hbm,
             src_v, dst_v, bufA, bufB, zbuf_v, nxi_v, acc_sh,
             semA, semB, semW0, semW1):
    s = lax.axis_index("s")
    c = lax.axis_index("c")
    w = s * NC + c
    dummy_grp = pts_hbm.at[pl.ds(0, GRP * BLK)]

    def zrow(i, _):
        for jj in range(4):
            zbuf_v[i, pl.ds(jj * 16, 16)] = jnp.zeros((16,), jnp.float32)
        return 0
    lax.fori_loop(0, ZROWS, zrow, 0)

    def zero_region(reg):
        def zacc(m, _):
            pltpu.sync_copy(
                zbuf_v,
                acc_sh.at[pl.ds((s * NREG + reg) * ACC_ROWS + m * ZROWS,
                                ZROWS)])
            return 0
        lax.fori_loop(0, ACC_ROWS // ZROWS, zacc, 0)

    zero_region(0)
    zero_region(1)
    semW = (semW0, semW1)

    def fire_group(g, buf, sem, cbase):
        for k in range(GRP):
            pltpu.async_copy(pts_hbm.at[src_v.at[cbase + g * GRP + k]],
                             buf.at[pl.ds(k * BLK, BLK)], sem)

    for chunk in range(NCHUNK):
        reg = chunk % NREG
        accbase = (s * NREG + reg) * ACC_ROWS
        cbase = 0
        pltpu.sync_copy(src_hbm.at[pl.ds((w * NCHUNK + chunk) * NBLK, NBLK)],
                        src_v)
        pltpu.sync_copy(dst_hbm.at[pl.ds((w * NCHUNK + chunk) * NBLK, NBLK)],
                        dst_v)
        orow = (w * NCHUNK + chunk) * PLANE_ROWS
        wb = [(acc_sh.at[pl.ds(accbase + q * PLANE_ROWS, PLANE_ROWS)],
               fq.at[pl.ds(orow, PLANE_ROWS)])
              for q, fq in enumerate((f0_hbm, f1_hbm, f2_hbm))]
        if chunk >= NREG:
            for a, d in wb:
                pltpu.make_async_copy(a, d, semW[reg]).wait()
            zero_region(reg)

        fire_group(0, bufA, semA, cbase)

        def pair_body(i, _):
            gA = 2 * i
            pltpu.make_async_copy(dummy_grp, bufA, semA).wait()
            fire_group(gA + 1, bufB, semB, cbase)
            for k in range(GRP):
                pltpu.sync_copy(bufA.at[pl.ds(k * BLK, BLK)],
                                acc_sh.at[dst_v.at[cbase + gA * GRP + k]],
                                add=True)
            pltpu.make_async_copy(dummy_grp, bufB, semB).wait()
            fire_group(lax.rem(gA + 2, NGRP), bufA, semA, cbase)
            for k in range(GRP):
                pltpu.sync_copy(bufB.at[pl.ds(k * BLK, BLK)],
                                acc_sh.at[dst_v.at[cbase + (gA + 1) * GRP + k]],
                                add=True)
            return 0
        lax.fori_loop(0, NGRP // 2, pair_body, 0)
        pltpu.make_async_copy(dummy_grp, bufA, semA).wait()

        for a, d in wb:
            pltpu.async_copy(a, d, semW[reg])

    for chunk in range(NCHUNK - NREG, NCHUNK):
        reg = chunk % NREG
        accbase = (s * NREG + reg) * ACC_ROWS
        orow = (w * NCHUNK + chunk) * PLANE_ROWS
        for q, fq in enumerate((f0_hbm, f1_hbm, f2_hbm)):
            pltpu.make_async_copy(
                acc_sh.at[pl.ds(accbase + q * PLANE_ROWS, PLANE_ROWS)],
                fq.at[pl.ds(orow, PLANE_ROWS)], semW[reg]).wait()

    pltpu.sync_copy(nx_hbm.at[pl.ds(w * NXBLK, NXBLK)], nxi_v)

    def nx_body(m, _):
        pltpu.async_copy(pts_hbm.at[nxi_v.at[m]],
                         bufA.at[pl.ds(0, BLK)], semA).wait()
        pltpu.sync_copy(bufA.at[pl.ds(0, BLK)],
                        nxr_hbm.at[pl.ds(w * NXPW + m * BLK, BLK)])
        return 0
    lax.fori_loop(0, NXBLK, nx_body, 0)


def _sc_gather_scatter(pts_flat, src_idx, dst_idx, nx_idx):
    mesh = plsc.VectorSubcoreMesh(core_axis_name="c", subcore_axis_name="s")
    fn = pl.kernel(
        _sc_body,
        out_type=[
            jax.ShapeDtypeStruct((NCENT * 2, CIN), jnp.float32),
            jax.ShapeDtypeStruct((NCENT * 2, CIN), jnp.float32),
            jax.ShapeDtypeStruct((NCENT * 2, CIN), jnp.float32),
            jax.ShapeDtypeStruct((NCENT, CIN), jnp.float32),
        ],
        mesh=mesh,
        scratch_types=[
            pltpu.VMEM((NBLK, BLK), jnp.int32),
            pltpu.VMEM((NBLK, BLK), jnp.int32),
            pltpu.VMEM((GRP * BLK, CIN), jnp.float32),
            pltpu.VMEM((GRP * BLK, CIN), jnp.float32),
            pltpu.VMEM((ZROWS, CIN), jnp.float32),
            pltpu.VMEM((NXBLK, BLK), jnp.int32),
            pltpu.VMEM_SHARED((NS * NREG * ACC_ROWS, CIN), jnp.float32),
            pltpu.SemaphoreType.DMA,
            pltpu.SemaphoreType.DMA,
            pltpu.SemaphoreType.DMA,
            pltpu.SemaphoreType.DMA,
        ],
        compiler_params=pltpu.CompilerParams(use_tc_tiling_on_sc=False),
    )
    return fn(pts_flat, src_idx, dst_idx, nx_idx)


def _head_kernel(a0, a1, a2, w0, w1, w2, bc, g1, be1, wl, bl, g2, be2, out):
    n = jnp.float32(NCENT)
    x = (lax.dot_general(a0[...], w0[...], (((1,), (1,)), ((), ())),
                         preferred_element_type=jnp.float32)
         + lax.dot_general(a1[...], w1[...], (((1,), (1,)), ((), ())),
                           preferred_element_type=jnp.float32)
         + lax.dot_general(a2[...], w2[...], (((1,), (1,)), ((), ())),
                           preferred_element_type=jnp.float32)
         + bc[...])
    mu = jnp.sum(x, axis=0, keepdims=True) / n
    var = jnp.sum(x * x, axis=0, keepdims=True) / n - mu * mu
    x = (x - mu) / jnp.sqrt(var + EPS) * g1[...] + be1[...]
    x = jnp.maximum(x, 0.0)
    x = lax.dot_general(x, wl[...], (((1,), (1,)), ((), ())),
                        preferred_element_type=jnp.float32) + bl[...]
    mu2 = jnp.sum(x, axis=0, keepdims=True) / n
    var2 = jnp.sum(x * x, axis=0, keepdims=True) / n - mu2 * mu2
    x = (x - mu2) / jnp.sqrt(var2 + EPS) * g2[...] + be2[...]
    out[...] = jnp.maximum(x, 0.0)


def _head(planes, W_conv, b_conv, gamma1, beta1, W_lin, b_lin, gamma2, beta2):
    wpad = jnp.pad(W_conv, ((0, 0), (0, NQ * 128 - P * CIN)))
    ws = [wpad[:, q * 128:(q + 1) * 128] for q in range(NQ)]
    return pl.pallas_call(
        _head_kernel,
        out_shape=jax.ShapeDtypeStruct((NCENT, COUT), jnp.float32),
    )(planes[0], planes[1], planes[2], ws[0], ws[1], ws[2],
      b_conv.reshape(1, COUT), gamma1.reshape(1, COUT),
      beta1.reshape(1, COUT), W_lin, b_lin.reshape(1, COUT),
      gamma2.reshape(1, COUT), beta2.reshape(1, COUT))


@jax.jit
def _run(xyz, points, local_coordinates, neighbor_lists, data_idx,
         W_conv, b_conv, gamma1, beta1, W_lin, b_lin, gamma2, beta2):
    pts_flat = jnp.concatenate([points, xyz], axis=2).reshape(B * N, CIN)
    src_idx, dst_idx, nx_idx = _make_indices(
        local_coordinates, neighbor_lists, data_idx)
    f0, f1, f2, nx_rows = _sc_gather_scatter(pts_flat, src_idx, dst_idx,
                                             nx_idx)
    planes = [f.reshape(NCENT, 2 * CIN) for f in (f0, f1, f2)]
    out = _head(planes, W_conv, b_conv, gamma1, beta1, W_lin, b_lin,
                gamma2, beta2)
    new_xyz = nx_rows[:, CIN - 3:].reshape(B, NP, 3)
    new_points = out.reshape(B, NP, COUT)
    return new_xyz, new_points


def kernel(xyz, points, local_coordinates, neighbor_lists, parameter_list,
           data_idx, W_conv, b_conv, gamma1, beta1, W_lin, b_lin,
           gamma2, beta2):
    return _run(xyz, points, local_coordinates, neighbor_lists, data_idx,
                W_conv, b_conv, gamma1, beta1, W_lin, b_lin, gamma2, beta2)


# speedup vs baseline: 1.1325x; 1.0474x over previous
import functools
import jax
import jax.numpy as jnp
from jax import lax
from jax.experimental import pallas as pl
from jax.experimental.pallas import tpu as pltpu, tpu_sc as plsc

B, N, NP, K, CIN, COUT, P = 8, 8192, 2048, 32, 64, 64, 5
RADIUS = 1.5
EPS = 1e-5

NC, NS = 2, 16
NW = NC * NS
NCENT = B * NP
CPW = NCENT // NW
CHUNK_C = 64
NCHUNK = CPW // CHUNK_C
ROWS_PER_CHUNK = CHUNK_C * K
BLK = 128
NBLK = ROWS_PER_CHUNK // BLK
GRP = 2
NGRP = NBLK // GRP
NQ = 3
ACC_ROWS = CHUNK_C * 2 * NQ
PLANE_ROWS = CHUNK_C * 2
NREG = 2
ZROWS = 64
NXPW = NCENT // NW
NXBLK = NXPW // BLK


def _idx_kernel(xcol, ycol, neigh, didx, src_out, dst_out, nx_out):
    j = pl.program_id(0)
    rows = xcol.shape[0]
    base = j * rows * 128
    ii = (base
          + lax.broadcasted_iota(jnp.int32, (rows, 128), 0) * 128
          + lax.broadcasted_iota(jnp.int32, (rows, 128), 1))
    dist = jnp.sqrt(xcol[...] * xcol[...] + ycol[...] * ycol[...])
    dist = jnp.minimum(dist / RADIUS, 0.99)
    bins = jnp.floor(dist * P).astype(jnp.int32)
    b = ii >> 16
    src_out[...] = b * N + neigh[...]
    c = ii >> 5
    s = c >> 10
    reg = (c >> 6) & 1
    cl = c & (CHUNK_C - 1)
    dst_out[...] = ((s * NREG + reg) * ACC_ROWS + (bins >> 1) * PLANE_ROWS
                    + cl * 2 + (bins & 1))
    nrows = didx.shape[0]
    i2 = (j * nrows * 128
          + lax.broadcasted_iota(jnp.int32, (nrows, 128), 0) * 128
          + lax.broadcasted_iota(jnp.int32, (nrows, 128), 1))
    b2 = i2 >> 11
    nx_out[...] = b2 * N + didx[...]


def _make_indices(lc, neighbor_lists, data_idx):
    lcf = lc.reshape(NCENT * K, 3)
    xcol = lcf[:, 0].reshape(NCENT * K // 128, 128)
    ycol = lcf[:, 1].reshape(NCENT * K // 128, 128)
    neigh = neighbor_lists.reshape(NCENT * K // 128, 128).astype(jnp.int32)
    didx = data_idx.reshape(NCENT // 128, 128).astype(jnp.int32)
    G = 4
    rb = NCENT * K // 128 // G
    nb = NCENT // 128 // G
    src, dst, nx = pl.pallas_call(
        _idx_kernel,
        grid=(G,),
        in_specs=[
            pl.BlockSpec((rb, 128), lambda j: (j, 0)),
            pl.BlockSpec((rb, 128), lambda j: (j, 0)),
            pl.BlockSpec((rb, 128), lambda j: (j, 0)),
            pl.BlockSpec((nb, 128), lambda j: (j, 0)),
        ],
        out_specs=[
            pl.BlockSpec((rb, 128), lambda j: (j, 0)),
            pl.BlockSpec((rb, 128), lambda j: (j, 0)),
            pl.BlockSpec((nb, 128), lambda j: (j, 0)),
        ],
        out_shape=[
            jax.ShapeDtypeStruct((NCENT * K // 128, 128), jnp.int32),
            jax.ShapeDtypeStruct((NCENT * K // 128, 128), jnp.int32),
            jax.ShapeDtypeStruct((NCENT // 128, 128), jnp.int32),
        ],
    )(xcol, ycol, neigh, didx)
    return src, dst, nx


def _sc_body(pts_hbm, src_hbm, dst_hbm, nx_hbm, xyzp_hbm,
             f0_hbm, f1_hbm, f2_hbm, nxr_hbm,
             src_v, dst_v, bufA, bufB, zbuf_v, nxi_v, nxo_v, nxg_v, acc_sh,
             semA, semB, semW0, semW1):
    s = lax.axis_index("s")
    c = lax.axis_index("c")
    w = s * NC + c
    dummy_grp = pts_hbm.at[pl.ds(0, GRP * BLK)]

    def zrow(i, _):
        for jj in range(4):
            zbuf_v[i, pl.ds(jj * 16, 16)] = jnp.zeros((16,), jnp.float32)
        return 0
    lax.fori_loop(0, ZROWS, zrow, 0)

    def zero_region(reg):
        def zacc(m, _):
            pltpu.sync_copy(
                zbuf_v,
                acc_sh.at[pl.ds((s * NREG + reg) * ACC_ROWS + m * ZROWS,
                                ZROWS)])
            return 0
        lax.fori_loop(0, ACC_ROWS // ZROWS, zacc, 0)

    zero_region(0)
    zero_region(1)
    semW = (semW0, semW1)

    def fire_group(g, buf, sem, cbase):
        for k in range(GRP):
            pltpu.async_copy(pts_hbm.at[src_v.at[cbase + g * GRP + k]],
                             buf.at[pl.ds(k * BLK, BLK)], sem)

    for chunk in range(NCHUNK):
        reg = chunk % NREG
        accbase = (s * NREG + reg) * ACC_ROWS
        cbase = 0
        pltpu.sync_copy(src_hbm.at[pl.ds((w * NCHUNK + chunk) * NBLK, NBLK)],
                        src_v)
        pltpu.sync_copy(dst_hbm.at[pl.ds((w * NCHUNK + chunk) * NBLK, NBLK)],
                        dst_v)
        orow = (w * NCHUNK + chunk) * PLANE_ROWS
        wb = [(acc_sh.at[pl.ds(accbase + q * PLANE_ROWS, PLANE_ROWS)],
               fq.at[pl.ds(orow, PLANE_ROWS)])
              for q, fq in enumerate((f0_hbm, f1_hbm, f2_hbm))]
        if chunk >= NREG:
            for a, d in wb:
                pltpu.make_async_copy(a, d, semW[reg]).wait()
            zero_region(reg)

        fire_group(0, bufA, semA, cbase)

        def pair_body(i, _):
            gA = 2 * i
            pltpu.make_async_copy(dummy_grp, bufA, semA).wait()
            fire_group(gA + 1, bufB, semB, cbase)
            for k in range(GRP):
                pltpu.sync_copy(bufA.at[pl.ds(k * BLK, BLK)],
                                acc_sh.at[dst_v.at[cbase + gA * GRP + k]],
                                add=True)
            pltpu.make_async_copy(dummy_grp, bufB, semB).wait()
            fire_group(lax.rem(gA + 2, NGRP), bufA, semA, cbase)
            for k in range(GRP):
                pltpu.sync_copy(bufB.at[pl.ds(k * BLK, BLK)],
                                acc_sh.at[dst_v.at[cbase + (gA + 1) * GRP + k]],
                                add=True)
            return 0
        lax.fori_loop(0, NGRP // 2, pair_body, 0)
        pltpu.make_async_copy(dummy_grp, bufA, semA).wait()

        for a, d in wb:
            pltpu.async_copy(a, d, semW[reg])

    for chunk in range(NCHUNK - NREG, NCHUNK):
        reg = chunk % NREG
        accbase = (s * NREG + reg) * ACC_ROWS
        orow = (w * NCHUNK + chunk) * PLANE_ROWS
        for q, fq in enumerate((f0_hbm, f1_hbm, f2_hbm)):
            pltpu.make_async_copy(
                acc_sh.at[pl.ds(accbase + q * PLANE_ROWS, PLANE_ROWS)],
                fq.at[pl.ds(orow, PLANE_ROWS)], semW[reg]).wait()

    pltpu.sync_copy(nx_hbm.at[pl.ds(w * NXBLK, NXBLK)], nxi_v)
    for d in range(3):
        off = d * (B * N)

        def addoff(m, _):
            for jj in range(8):
                sl = pl.ds(jj * 16, 16)
                nxo_v[m, sl] = nxi_v[m, sl] + off
            return 0
        lax.fori_loop(0, NXBLK, addoff, 0)

        def nx_body(m, _):
            pltpu.async_copy(xyzp_hbm.at[nxo_v.at[m]],
                             nxg_v.at[m], semA).wait()
            return 0
        lax.fori_loop(0, NXBLK, nx_body, 0)
        pltpu.sync_copy(nxg_v, nxr_hbm.at[d, pl.ds(w * NXBLK, NXBLK)])


def _sc_gather_scatter(pts_flat, src_idx, dst_idx, nx_idx, xyz_planar):
    mesh = plsc.VectorSubcoreMesh(core_axis_name="c", subcore_axis_name="s")
    fn = pl.kernel(
        _sc_body,
        out_type=[
            jax.ShapeDtypeStruct((NCENT * 2, CIN), jnp.float32),
            jax.ShapeDtypeStruct((NCENT * 2, CIN), jnp.float32),
            jax.ShapeDtypeStruct((NCENT * 2, CIN), jnp.float32),
            jax.ShapeDtypeStruct((3, NCENT // BLK, BLK), jnp.float32),
        ],
        mesh=mesh,
        scratch_types=[
            pltpu.VMEM((NBLK, BLK), jnp.int32),
            pltpu.VMEM((NBLK, BLK), jnp.int32),
            pltpu.VMEM((GRP * BLK, CIN), jnp.float32),
            pltpu.VMEM((GRP * BLK, CIN), jnp.float32),
            pltpu.VMEM((ZROWS, CIN), jnp.float32),
            pltpu.VMEM((NXBLK, BLK), jnp.int32),
            pltpu.VMEM((NXBLK, BLK), jnp.int32),
            pltpu.VMEM((NXBLK, BLK), jnp.float32),
            pltpu.VMEM_SHARED((NS * NREG * ACC_ROWS, CIN), jnp.float32),
            pltpu.SemaphoreType.DMA,
            pltpu.SemaphoreType.DMA,
            pltpu.SemaphoreType.DMA,
            pltpu.SemaphoreType.DMA,
        ],
        compiler_params=pltpu.CompilerParams(use_tc_tiling_on_sc=False),
    )
    return fn(pts_flat, src_idx, dst_idx, nx_idx, xyz_planar)


def _head_kernel(a0, a1, a2, w0, w1, w2, bc, g1, be1, wl, bl, g2, be2, out):
    n = jnp.float32(NCENT)
    x = (lax.dot_general(a0[...], w0[...], (((1,), (1,)), ((), ())),
                         preferred_element_type=jnp.float32)
         + lax.dot_general(a1[...], w1[...], (((1,), (1,)), ((), ())),
                           preferred_element_type=jnp.float32)
         + lax.dot_general(a2[...], w2[...], (((1,), (1,)), ((), ())),
                           preferred_element_type=jnp.float32)
         + bc[...])
    mu = jnp.sum(x, axis=0, keepdims=True) / n
    var = jnp.sum(x * x, axis=0, keepdims=True) / n - mu * mu
    x = (x - mu) / jnp.sqrt(var + EPS) * g1[...] + be1[...]
    x = jnp.maximum(x, 0.0)
    x = lax.dot_general(x, wl[...], (((1,), (1,)), ((), ())),
                        preferred_element_type=jnp.float32) + bl[...]
    mu2 = jnp.sum(x, axis=0, keepdims=True) / n
    var2 = jnp.sum(x * x, axis=0, keepdims=True) / n - mu2 * mu2
    x = (x - mu2) / jnp.sqrt(var2 + EPS) * g2[...] + be2[...]
    out[...] = jnp.maximum(x, 0.0)


def _head(planes, W_conv, b_conv, gamma1, beta1, W_lin, b_lin, gamma2, beta2):
    wpad = jnp.pad(W_conv, ((0, 0), (0, NQ * 128 - P * CIN)))
    ws = [wpad[:, q * 128:(q + 1) * 128] for q in range(NQ)]
    return pl.pallas_call(
        _head_kernel,
        out_shape=jax.ShapeDtypeStruct((NCENT, COUT), jnp.float32),
    )(planes[0], planes[1], planes[2], ws[0], ws[1], ws[2],
      b_conv.reshape(1, COUT), gamma1.reshape(1, COUT),
      beta1.reshape(1, COUT), W_lin, b_lin.reshape(1, COUT),
      gamma2.reshape(1, COUT), beta2.reshape(1, COUT))


@jax.jit
def _run(xyz, points, local_coordinates, neighbor_lists, data_idx,
         W_conv, b_conv, gamma1, beta1, W_lin, b_lin, gamma2, beta2):
    pts_flat = jnp.concatenate([points, xyz], axis=2).reshape(B * N, CIN)
    src_idx, dst_idx, nx_idx = _make_indices(
        local_coordinates, neighbor_lists, data_idx)
    xyz_planar = jnp.moveaxis(xyz, 2, 0).reshape(3 * B * N)
    f0, f1, f2, nx_pl = _sc_gather_scatter(pts_flat, src_idx, dst_idx,
                                           nx_idx, xyz_planar)
    planes = [f.reshape(NCENT, 2 * CIN) for f in (f0, f1, f2)]
    out = _head(planes, W_conv, b_conv, gamma1, beta1, W_lin, b_lin,
                gamma2, beta2)
    new_xyz = jnp.moveaxis(nx_pl.reshape(3, B, NP), 0, 2)
    new_points = out.reshape(B, NP, COUT)
    return new_xyz, new_points


def kernel(xyz, points, local_coordinates, neighbor_lists, parameter_list,
           data_idx, W_conv, b_conv, gamma1, beta1, W_lin, b_lin,
           gamma2, beta2):
    return _run(xyz, points, local_coordinates, neighbor_lists, data_idx,
                W_conv, b_conv, gamma1, beta1, W_lin, b_lin, gamma2, beta2)
